# baseline (device time: 623893 ns/iter reference)
import jax
import jax.numpy as jnp
from jax import lax
from jax.experimental import pallas as pl
from jax.experimental.pallas import tpu as pltpu

N_DEV = 4
TS = 1024
NT = 4


def _fused(x, w_mat, m_per):
    m, k_per = x.shape
    n = w_mat.shape[1]
    half = n // 2
    assert half == NT * TS
    n_hops = N_DEV - 1

    def body(x_ref, w_ref, out_ref, recv_cw, recv_ccw, xv, wt, s_cw, s_ccw,
             vb, send_sems_cw, send_sems_ccw, recv_sems_cw, recv_sems_ccw,
             dma_a, dma_b):
        my = lax.axis_index("i")
        left = lax.rem(my + N_DEV - 1, N_DEV)
        right = lax.rem(my + 1, N_DEV)

        cpx = pltpu.make_async_copy(x_ref, xv, dma_a)
        cpx.start()

        barrier_sem = pltpu.get_barrier_semaphore()
        for nbr in (left, right):
            pl.semaphore_signal(
                barrier_sem, inc=1,
                device_id=(nbr,), device_id_type=pl.DeviceIdType.MESH,
            )
        pl.semaphore_wait(barrier_sem, 2)
        cpx.wait()

        def copy(src, dst, sem):
            cp = pltpu.make_async_copy(src, dst, sem)
            cp.start()
            return cp

        dirs = (
            (s_cw, recv_cw, send_sems_cw, recv_sems_cw, 0, right),
            (s_ccw, recv_ccw, send_sems_ccw, recv_sems_ccw, half, left),
        )
        descs = [[], []]

        for h in range(n_hops):
            c_cw = lax.rem(my + 2 * N_DEV - 1 - h, N_DEV)
            c_ccw = lax.rem(my + 1 + h, N_DEV)
            slot = h % 2
            prev = (h - 1) % 2
            for t in range(NT):
                for d, ((s_buf, r_buf, s_sems, r_sems, col0, peer), c) in (
                        enumerate(zip(dirs, (c_cw, c_ccw)))):
                    kk = h * NT + t
                    sslot = kk % 2
                    if kk >= 2:
                        descs[d][kk - 2].wait_send()
                    cpw = copy(w_ref.at[:, pl.ds(col0 + t * TS, TS)],
                               wt, dma_a)
                    if h > 0:
                        descs[d][kk - NT].wait_recv()
                        cpr = copy(r_buf.at[prev, t], vb, dma_b)
                    cpw.wait()
                    xa = xv[pl.ds(c * m_per, m_per), :]
                    if h > 0:
                        cpr.wait()
                        s_buf[sslot] = jnp.dot(
                            xa, wt[...], preferred_element_type=jnp.float32
                        ) + vb[...]
                    else:
                        s_buf[sslot] = jnp.dot(
                            xa, wt[...], preferred_element_type=jnp.float32
                        )
                    rdma = pltpu.make_async_remote_copy(
                        src_ref=s_buf.at[sslot],
                        dst_ref=r_buf.at[slot, t],
                        send_sem=s_sems.at[sslot],
                        recv_sem=r_sems.at[slot, t],
                        device_id=(peer,),
                        device_id_type=pl.DeviceIdType.MESH,
                    )
                    rdma.start()
                    descs[d].append(rdma)

        last = (n_hops - 1) % 2
        for t in range(NT):
            for d, (s_buf, r_buf, s_sems, r_sems, col0, peer) in (
                    enumerate(dirs)):
                kk = n_hops * NT + t
                if kk - 2 < len(descs[d]):
                    descs[d][kk - 2].wait_send()
                descs[d][(n_hops - 1) * NT + t].wait_recv()
                cols = pl.ds(col0 + t * TS, TS)
                cpw = copy(w_ref.at[:, cols], wt, dma_a)
                cpr = copy(r_buf.at[last, t], vb, dma_b)
                cpw.wait()
                cpr.wait()
                xa = xv[pl.ds(my * m_per, m_per), :]
                y = jnp.dot(
                    xa, wt[...], preferred_element_type=jnp.float32
                ) + vb[...]
                vb[...] = y * jax.nn.sigmoid(y)
                copy(vb, out_ref.at[:, cols], dma_b).wait()

    out, _, _ = pl.pallas_call(
        body,
        out_shape=[
            jax.ShapeDtypeStruct((m_per, n), jnp.float32),
            jax.ShapeDtypeStruct((2, NT, m_per, TS), jnp.float32),
            jax.ShapeDtypeStruct((2, NT, m_per, TS), jnp.float32),
        ],
        in_specs=[
            pl.BlockSpec(memory_space=pl.ANY),
            pl.BlockSpec(memory_space=pl.ANY),
        ],
        out_specs=[
            pl.BlockSpec(memory_space=pl.ANY),
            pl.BlockSpec(memory_space=pl.ANY),
            pl.BlockSpec(memory_space=pl.ANY),
        ],
        scratch_shapes=[
            pltpu.VMEM((m, k_per), jnp.float32),
            pltpu.VMEM((k_per, TS), jnp.float32),
            pltpu.VMEM((2, m_per, TS), jnp.float32),
            pltpu.VMEM((2, m_per, TS), jnp.float32),
            pltpu.VMEM((m_per, TS), jnp.float32),
            pltpu.SemaphoreType.DMA((2,)),
            pltpu.SemaphoreType.DMA((2,)),
            pltpu.SemaphoreType.DMA((2, NT)),
            pltpu.SemaphoreType.DMA((2, NT)),
            pltpu.SemaphoreType.DMA,
            pltpu.SemaphoreType.DMA,
        ],
        compiler_params=pltpu.CompilerParams(
            collective_id=0, vmem_limit_bytes=63 * 1024 * 1024,
        ),
    )(x, w_mat)
    return out


def kernel(x, w_mat):
    m = x.shape[0]
    m_per = m // N_DEV
    return _fused(x, w_mat, m_per)


# device time: 605208 ns/iter; 1.0309x vs baseline; 1.0309x over previous
import jax
import jax.numpy as jnp
from jax import lax
from jax.experimental import pallas as pl
from jax.experimental.pallas import tpu as pltpu

N_DEV = 4
TS = 1024
NT = 4
SD = 3


def _fused(x, w_mat, m_per):
    m, k_per = x.shape
    n = w_mat.shape[1]
    half = n // 2
    assert half == NT * TS
    n_hops = N_DEV - 1

    def body(x_ref, w_ref, out_ref, recv_cw, recv_ccw, xt_cw, xt_ccw, wt,
             s_cw, s_ccw, vb, send_sems_cw, send_sems_ccw, recv_sems_cw,
             recv_sems_ccw, dma_x, dma_w, dma_r):
        my = lax.axis_index("i")
        left = lax.rem(my + N_DEV - 1, N_DEV)
        right = lax.rem(my + 1, N_DEV)

        barrier_sem = pltpu.get_barrier_semaphore()
        for nbr in (left, right):
            pl.semaphore_signal(
                barrier_sem, inc=1,
                device_id=(nbr,), device_id_type=pl.DeviceIdType.MESH,
            )
        pl.semaphore_wait(barrier_sem, 2)

        def copy(src, dst, sem):
            cp = pltpu.make_async_copy(src, dst, sem)
            cp.start()
            return cp

        dirs = (
            (xt_cw, s_cw, recv_cw, send_sems_cw, recv_sems_cw, 0, right),
            (xt_ccw, s_ccw, recv_ccw, send_sems_ccw, recv_sems_ccw, half, left),
        )
        descs = [[], []]

        for h in range(n_hops):
            c_cw = lax.rem(my + 2 * N_DEV - 1 - h, N_DEV)
            c_ccw = lax.rem(my + 1 + h, N_DEV)
            slot = h % 2
            prev = (h - 1) % 2
            for t in range(NT):
                for d, ((xt, s_buf, r_buf, s_sems, r_sems, col0, peer), c) in (
                        enumerate(zip(dirs, (c_cw, c_ccw)))):
                    kk = h * NT + t
                    sslot = kk % SD
                    if t == 0:
                        cpx = copy(x_ref.at[pl.ds(c * m_per, m_per), :],
                                   xt, dma_x)
                    cpw = copy(w_ref.at[:, pl.ds(col0 + t * TS, TS)],
                               wt, dma_w)
                    if kk >= SD:
                        descs[d][kk - SD].wait_send()
                    if h > 0:
                        descs[d][kk - NT].wait_recv()
                        cpr = copy(r_buf.at[prev, t], vb, dma_r)
                    if t == 0:
                        cpx.wait()
                    cpw.wait()
                    if h > 0:
                        cpr.wait()
                        s_buf[sslot] = jnp.dot(
                            xt[...], wt[...],
                            preferred_element_type=jnp.float32,
                        ) + vb[...]
                    else:
                        s_buf[sslot] = jnp.dot(
                            xt[...], wt[...],
                            preferred_element_type=jnp.float32,
                        )
                    rdma = pltpu.make_async_remote_copy(
                        src_ref=s_buf.at[sslot],
                        dst_ref=r_buf.at[slot, t],
                        send_sem=s_sems.at[sslot],
                        recv_sem=r_sems.at[slot, t],
                        device_id=(peer,),
                        device_id_type=pl.DeviceIdType.MESH,
                    )
                    rdma.start()
                    descs[d].append(rdma)

        last = (n_hops - 1) % 2
        cpx0 = copy(x_ref.at[pl.ds(my * m_per, m_per), :], xt_cw, dma_x)
        cpx0.wait()
        for t in range(NT):
            for d, (xt, s_buf, r_buf, s_sems, r_sems, col0, peer) in (
                    enumerate(dirs)):
                kk = n_hops * NT + t
                if kk - SD < len(descs[d]):
                    descs[d][kk - SD].wait_send()
                descs[d][(n_hops - 1) * NT + t].wait_recv()
                cols = pl.ds(col0 + t * TS, TS)
                cpw = copy(w_ref.at[:, cols], wt, dma_w)
                cpr = copy(r_buf.at[last, t], vb, dma_r)
                cpw.wait()
                cpr.wait()
                y = jnp.dot(
                    xt_cw[...], wt[...], preferred_element_type=jnp.float32
                ) + vb[...]
                vb[...] = y * jax.nn.sigmoid(y)
                copy(vb, out_ref.at[:, cols], dma_r).wait()

    out, _, _ = pl.pallas_call(
        body,
        out_shape=[
            jax.ShapeDtypeStruct((m_per, n), jnp.float32),
            jax.ShapeDtypeStruct((2, NT, m_per, TS), jnp.float32),
            jax.ShapeDtypeStruct((2, NT, m_per, TS), jnp.float32),
        ],
        in_specs=[
            pl.BlockSpec(memory_space=pl.ANY),
            pl.BlockSpec(memory_space=pl.ANY),
        ],
        out_specs=[
            pl.BlockSpec(memory_space=pl.ANY),
            pl.BlockSpec(memory_space=pl.ANY),
            pl.BlockSpec(memory_space=pl.ANY),
        ],
        scratch_shapes=[
            pltpu.VMEM((m_per, k_per), jnp.float32),
            pltpu.VMEM((m_per, k_per), jnp.float32),
            pltpu.VMEM((k_per, TS), jnp.float32),
            pltpu.VMEM((SD, m_per, TS), jnp.float32),
            pltpu.VMEM((SD, m_per, TS), jnp.float32),
            pltpu.VMEM((m_per, TS), jnp.float32),
            pltpu.SemaphoreType.DMA((SD,)),
            pltpu.SemaphoreType.DMA((SD,)),
            pltpu.SemaphoreType.DMA((2, NT)),
            pltpu.SemaphoreType.DMA((2, NT)),
            pltpu.SemaphoreType.DMA,
            pltpu.SemaphoreType.DMA,
            pltpu.SemaphoreType.DMA,
        ],
        compiler_params=pltpu.CompilerParams(
            collective_id=0, vmem_limit_bytes=63 * 1024 * 1024,
        ),
    )(x, w_mat)
    return out


def kernel(x, w_mat):
    m = x.shape[0]
    m_per = m // N_DEV
    return _fused(x, w_mat, m_per)


# device time: 597727 ns/iter; 1.0438x vs baseline; 1.0125x over previous
import jax
import jax.numpy as jnp
from jax import lax
from jax.experimental import pallas as pl
from jax.experimental.pallas import tpu as pltpu

N_DEV = 4
TS = 1024
NT = 4
SD = 3


def _fused(x, w_mat, m_per):
    m, k_per = x.shape
    n = w_mat.shape[1]
    half = n // 2
    assert half == NT * TS
    n_hops = N_DEV - 1

    def body(x_ref, w_ref, out_ref, recv_cw, recv_ccw, xt_cw, xt_ccw, wt,
             s_cw, s_ccw, vb, vb2, send_sems_cw, send_sems_ccw, recv_sems_cw,
             recv_sems_ccw, dma_x, dma_w, dma_r):
        my = lax.axis_index("i")
        left = lax.rem(my + N_DEV - 1, N_DEV)
        right = lax.rem(my + 1, N_DEV)

        barrier_sem = pltpu.get_barrier_semaphore()
        for nbr in (left, right):
            pl.semaphore_signal(
                barrier_sem, inc=1,
                device_id=(nbr,), device_id_type=pl.DeviceIdType.MESH,
            )
        pl.semaphore_wait(barrier_sem, 2)

        def copy(src, dst, sem):
            cp = pltpu.make_async_copy(src, dst, sem)
            cp.start()
            return cp

        dirs = (
            (xt_cw, s_cw, recv_cw, send_sems_cw, recv_sems_cw, 0, right),
            (xt_ccw, s_ccw, recv_ccw, send_sems_ccw, recv_sems_ccw, half, left),
        )
        descs = [[], []]

        for h in range(n_hops):
            c_cw = lax.rem(my + 2 * N_DEV - 1 - h, N_DEV)
            c_ccw = lax.rem(my + 1 + h, N_DEV)
            slot = h % 2
            prev = (h - 1) % 2
            for t in range(NT):
                for d, ((xt, s_buf, r_buf, s_sems, r_sems, col0, peer), c) in (
                        enumerate(zip(dirs, (c_cw, c_ccw)))):
                    kk = h * NT + t
                    sslot = kk % SD
                    if t == 0:
                        cpx = copy(x_ref.at[pl.ds(c * m_per, m_per), :],
                                   xt, dma_x)
                    cpw = copy(w_ref.at[:, pl.ds(col0 + t * TS, TS)],
                               wt, dma_w)
                    if kk >= SD:
                        descs[d][kk - SD].wait_send()
                    if h > 0:
                        descs[d][kk - NT].wait_recv()
                        cpr = copy(r_buf.at[prev, t], vb, dma_r)
                    if t == 0:
                        cpx.wait()
                    cpw.wait()
                    if h > 0:
                        cpr.wait()
                        s_buf[sslot] = jnp.dot(
                            xt[...], wt[...],
                            preferred_element_type=jnp.float32,
                        ) + vb[...]
                    else:
                        s_buf[sslot] = jnp.dot(
                            xt[...], wt[...],
                            preferred_element_type=jnp.float32,
                        )
                    rdma = pltpu.make_async_remote_copy(
                        src_ref=s_buf.at[sslot],
                        dst_ref=r_buf.at[slot, t],
                        send_sem=s_sems.at[sslot],
                        recv_sem=r_sems.at[slot, t],
                        device_id=(peer,),
                        device_id_type=pl.DeviceIdType.MESH,
                    )
                    rdma.start()
                    descs[d].append(rdma)

        cpx0 = copy(x_ref.at[pl.ds(my * m_per, m_per), :], xt_cw, dma_x)
        cpx0.wait()
        for t in range(NT):
            for d, (xt, s_buf, r_buf, s_sems, r_sems, col0, peer) in (
                    enumerate(dirs)):
                cols = pl.ds(col0 + t * TS, TS)
                copy(w_ref.at[:, cols], wt, dma_w).wait()
                vb[...] = jnp.dot(
                    xt_cw[...], wt[...], preferred_element_type=jnp.float32
                )
                copy(vb, out_ref.at[:, cols], dma_r).wait()

        last = (n_hops - 1) % 2
        for t in range(NT):
            for d, (xt, s_buf, r_buf, s_sems, r_sems, col0, peer) in (
                    enumerate(dirs)):
                kk = n_hops * NT + t
                if kk - SD < len(descs[d]):
                    descs[d][kk - SD].wait_send()
                descs[d][(n_hops - 1) * NT + t].wait_recv()
                cols = pl.ds(col0 + t * TS, TS)
                cpo = copy(out_ref.at[:, cols], vb2, dma_w)
                cpr = copy(r_buf.at[last, t], vb, dma_r)
                cpo.wait()
                cpr.wait()
                y = vb2[...] + vb[...]
                vb[...] = y * jax.nn.sigmoid(y)
                copy(vb, out_ref.at[:, cols], dma_r).wait()

    out, _, _ = pl.pallas_call(
        body,
        out_shape=[
            jax.ShapeDtypeStruct((m_per, n), jnp.float32),
            jax.ShapeDtypeStruct((2, NT, m_per, TS), jnp.float32),
            jax.ShapeDtypeStruct((2, NT, m_per, TS), jnp.float32),
        ],
        in_specs=[
            pl.BlockSpec(memory_space=pl.ANY),
            pl.BlockSpec(memory_space=pl.ANY),
        ],
        out_specs=[
            pl.BlockSpec(memory_space=pl.ANY),
            pl.BlockSpec(memory_space=pl.ANY),
            pl.BlockSpec(memory_space=pl.ANY),
        ],
        scratch_shapes=[
            pltpu.VMEM((m_per, k_per), jnp.float32),
            pltpu.VMEM((m_per, k_per), jnp.float32),
            pltpu.VMEM((k_per, TS), jnp.float32),
            pltpu.VMEM((SD, m_per, TS), jnp.float32),
            pltpu.VMEM((SD, m_per, TS), jnp.float32),
            pltpu.VMEM((m_per, TS), jnp.float32),
            pltpu.VMEM((m_per, TS), jnp.float32),
            pltpu.SemaphoreType.DMA((SD,)),
            pltpu.SemaphoreType.DMA((SD,)),
            pltpu.SemaphoreType.DMA((2, NT)),
            pltpu.SemaphoreType.DMA((2, NT)),
            pltpu.SemaphoreType.DMA,
            pltpu.SemaphoreType.DMA,
            pltpu.SemaphoreType.DMA,
        ],
        compiler_params=pltpu.CompilerParams(
            collective_id=0, vmem_limit_bytes=63 * 1024 * 1024,
        ),
    )(x, w_mat)
    return out


def kernel(x, w_mat):
    m = x.shape[0]
    m_per = m // N_DEV
    return _fused(x, w_mat, m_per)
